# Initial kernel scaffold; baseline (speedup 1.0000x reference)
import jax
import jax.numpy as jnp
from jax.experimental import pallas as pl


def kernel(cur_prev_aff, feat):
    raise NotImplementedError("write your pallas kernel here")


# speedup vs baseline: 22.8259x; 22.8259x over previous
import jax
import jax.numpy as jnp
from jax.experimental import pallas as pl

_TOPK = 10


def _assemble_body(aff_ref, feat_ref, out_ref):
    a = aff_ref[0]
    nb = a.shape[1]

    vals = a
    need = jnp.full((1, nb), float(_TOPK), dtype=jnp.float32)
    th = jnp.full((1, nb), -jnp.inf, dtype=jnp.float32)
    for _ in range(_TOPK):
        m = jnp.max(vals, axis=0, keepdims=True)
        ge = vals >= m
        c = jnp.sum(ge.astype(jnp.float32), axis=0, keepdims=True)
        th = jnp.where(need > 0.0, m, th)
        need = need - c
        vals = jnp.where(ge, -jnp.inf, vals)

    mx = jnp.max(a, axis=0, keepdims=True)
    e = jnp.where(a >= th, jnp.exp(a - mx), 0.0)
    s = jnp.sum(e, axis=0, keepdims=True)

    f = feat_ref[0]
    acc = jax.lax.dot_general(
        f, e, (((1,), (0,)), ((), ())), preferred_element_type=jnp.float32
    )
    out_ref[0] = acc * (1.0 / s)


def kernel(cur_prev_aff, feat):
    B, P, N = cur_prev_aff.shape
    C = feat.shape[1]
    NB = 512
    grid = (B, pl.cdiv(N, NB))
    return pl.pallas_call(
        _assemble_body,
        grid=grid,
        in_specs=[
            pl.BlockSpec((1, P, NB), lambda b, n: (b, 0, n)),
            pl.BlockSpec((1, C, P), lambda b, n: (b, 0, 0)),
        ],
        out_specs=pl.BlockSpec((1, C, NB), lambda b, n: (b, 0, n)),
        out_shape=jax.ShapeDtypeStruct((B, C, N), jnp.float32),
    )(cur_prev_aff, feat)


# speedup vs baseline: 38.6064x; 1.6913x over previous
import jax
import jax.numpy as jnp
from jax.experimental import pallas as pl

_TOPK = 10


_ROWS_PER_CHUNK = 8


def _assemble_body(aff_ref, feat_ref, out_ref):
    p = aff_ref.shape[1]
    nb = aff_ref.shape[2]
    s = _ROWS_PER_CHUNK

    def _insert(i, t):
        v = aff_ref[0, pl.ds(i * s, s), :]
        t = list(t)
        for j in range(_TOPK):
            hi = jnp.maximum(t[j], v)
            v = jnp.minimum(t[j], v)
            t[j] = hi
        return tuple(t)

    t0 = tuple(
        jnp.full((s, nb), -jnp.inf, dtype=jnp.float32) for _ in range(_TOPK)
    )
    t = jax.lax.fori_loop(0, p // s, _insert, t0, unroll=2)

    cand = jnp.concatenate(list(t), axis=0)
    vals = cand
    need = jnp.full((1, nb), float(_TOPK), dtype=jnp.float32)
    th = jnp.full((1, nb), -jnp.inf, dtype=jnp.float32)
    for _ in range(_TOPK):
        m = jnp.max(vals, axis=0, keepdims=True)
        ge = vals >= m
        c = jnp.sum(ge.astype(jnp.float32), axis=0, keepdims=True)
        th = jnp.where(need > 0.0, m, th)
        need = need - c
        vals = jnp.where(ge, -jnp.inf, vals)

    a = aff_ref[0]
    mx = jnp.max(t[0], axis=0, keepdims=True)
    e = jnp.where(a >= th, jnp.exp(a - mx), 0.0)
    s = jnp.sum(e, axis=0, keepdims=True)

    f = feat_ref[0]
    acc = jax.lax.dot_general(
        f, e, (((1,), (0,)), ((), ())), preferred_element_type=jnp.float32
    )
    out_ref[0] = acc * (1.0 / s)


def kernel(cur_prev_aff, feat):
    B, P, N = cur_prev_aff.shape
    C = feat.shape[1]
    NB = 512
    grid = (B, pl.cdiv(N, NB))
    return pl.pallas_call(
        _assemble_body,
        grid=grid,
        in_specs=[
            pl.BlockSpec((1, P, NB), lambda b, n: (b, 0, n)),
            pl.BlockSpec((1, C, P), lambda b, n: (b, 0, 0)),
        ],
        out_specs=pl.BlockSpec((1, C, NB), lambda b, n: (b, 0, n)),
        out_shape=jax.ShapeDtypeStruct((B, C, N), jnp.float32),
    )(cur_prev_aff, feat)


# speedup vs baseline: 44.8797x; 1.1625x over previous
import jax
import jax.numpy as jnp
from jax.experimental import pallas as pl

_TOPK = 10


_ROWS_PER_CHUNK = 8


def _assemble_body(aff_ref, feat_ref, out_ref):
    p = aff_ref.shape[1]
    nb = aff_ref.shape[2]
    s = _ROWS_PER_CHUNK

    def _insert(i, t):
        v = aff_ref[0, pl.ds(i * s, s), :]
        t = list(t)
        for j in range(_TOPK):
            hi = jnp.maximum(t[j], v)
            v = jnp.minimum(t[j], v)
            t[j] = hi
        return tuple(t)

    t0 = tuple(
        jnp.full((s, nb), -jnp.inf, dtype=jnp.float32) for _ in range(_TOPK)
    )
    t = jax.lax.fori_loop(0, p // s, _insert, t0, unroll=8)

    cand = jnp.concatenate(list(t), axis=0)
    vals = cand
    need = jnp.full((1, nb), float(_TOPK), dtype=jnp.float32)
    th = jnp.full((1, nb), -jnp.inf, dtype=jnp.float32)
    for _ in range(_TOPK):
        m = jnp.max(vals, axis=0, keepdims=True)
        ge = vals >= m
        c = jnp.sum(ge.astype(jnp.float32), axis=0, keepdims=True)
        th = jnp.where(need > 0.0, m, th)
        need = need - c
        vals = jnp.where(ge, -jnp.inf, vals)

    a = aff_ref[0]
    mx = jnp.max(t[0], axis=0, keepdims=True)
    e = jnp.where(a >= th, jnp.exp(a - mx), 0.0)
    s = jnp.sum(e, axis=0, keepdims=True)

    f = feat_ref[0]
    acc = jax.lax.dot_general(
        f, e, (((1,), (0,)), ((), ())), preferred_element_type=jnp.float32
    )
    out_ref[0] = acc * (1.0 / s)


def kernel(cur_prev_aff, feat):
    B, P, N = cur_prev_aff.shape
    C = feat.shape[1]
    NB = 512
    grid = (B, pl.cdiv(N, NB))
    return pl.pallas_call(
        _assemble_body,
        grid=grid,
        in_specs=[
            pl.BlockSpec((1, P, NB), lambda b, n: (b, 0, n)),
            pl.BlockSpec((1, C, P), lambda b, n: (b, 0, 0)),
        ],
        out_specs=pl.BlockSpec((1, C, NB), lambda b, n: (b, 0, n)),
        out_shape=jax.ShapeDtypeStruct((B, C, N), jnp.float32),
    )(cur_prev_aff, feat)
